# Initial kernel scaffold; baseline (speedup 1.0000x reference)
import jax
import jax.numpy as jnp
from jax.experimental import pallas as pl


def kernel(x, edge_index, edge_value, params):
    raise NotImplementedError("write your pallas kernel here")


# speedup vs baseline: 2.5219x; 2.5219x over previous
import functools

import jax
import jax.numpy as jnp
from jax import lax
from jax.experimental import pallas as pl
from jax.experimental.pallas import tpu as pltpu
from jax.experimental.pallas import tpu_sc as plsc

F32 = jnp.float32

E = 320000
N = 10000
NF = 64
W = 128
NWIN = E // W
NC, NS = 2, 16
NWORK = NC * NS
CH = 624
SC_ITERS = (NWIN + NWORK - 1) // NWORK

BE = 2000
GE = E // BE
BN = 2000
GN = N // BN
BH = 400
GH = N // BH

def _mesh():
    return plsc.VectorSubcoreMesh(core_axis_name="c", subcore_axis_name="s")


def _sc_scatter(mof, src2d, zeros64, zeros16, ones_w, with_counts):
    sums_t = jax.ShapeDtypeStruct((NC, N, 64), F32)
    out_type = [sums_t, jax.ShapeDtypeStruct((NC, N, 16), F32)] if with_counts else sums_t
    scratch = [
        pltpu.VMEM((W,), jnp.int32),
        pltpu.VMEM((W, 64), F32),
        pltpu.VMEM_SHARED((N, 64), F32),
    ]
    if with_counts:
        scratch += [pltpu.VMEM((W, 16), F32), pltpu.VMEM_SHARED((N, 16), F32)]

    @functools.partial(pl.kernel, mesh=_mesh(), out_type=out_type,
                       scratch_types=scratch)
    def scat(*refs):
        if with_counts:
            (mof_h, src_h, z64_h, z16_h, ones_h, outd_h, outc_h,
             idx_v, rows_v, accd, ones_v, accc) = refs
        else:
            mof_h, src_h, z64_h, outd_h, idx_v, rows_v, accd = refs
        c = lax.axis_index("c")
        s = lax.axis_index("s")
        wid = s * NC + c
        r0 = s * CH

        def _init(zh, acc):
            pltpu.sync_copy(zh.at[pl.ds(r0, CH), :], acc.at[pl.ds(r0, CH), :])

            @pl.when(s == NS - 1)
            def _():
                pltpu.sync_copy(zh.at[pl.ds(N - 16, 16), :],
                                acc.at[pl.ds(N - 16, 16), :])

        _init(z64_h, accd)
        if with_counts:
            _init(z16_h, accc)
            pltpu.sync_copy(ones_h, ones_v)
        plsc.subcore_barrier()

        def body(i, carry):
            widx = wid + NWORK * i

            @pl.when(widx < NWIN)
            def _():
                pltpu.sync_copy(src_h.at[pl.ds(widx * W, W)], idx_v)
                pltpu.sync_copy(mof_h.at[pl.ds(widx * W, W), :], rows_v)
                pltpu.sync_copy(rows_v, accd.at[idx_v], add=True)
                if with_counts:
                    pltpu.sync_copy(ones_v, accc.at[idx_v], add=True)
            return carry

        lax.fori_loop(0, SC_ITERS, body, 0)
        plsc.subcore_barrier()

        def _dump(acc, oh):
            pltpu.sync_copy(acc.at[pl.ds(r0, CH), :],
                            oh.at[c, pl.ds(r0, CH), :])

            @pl.when(s == NS - 1)
            def _():
                pltpu.sync_copy(acc.at[pl.ds(N - 16, 16), :],
                                oh.at[c, pl.ds(N - 16, 16), :])

        _dump(accd, outd_h)
        if with_counts:
            _dump(accc, outc_h)

    if with_counts:
        return scat(mof, src2d, zeros64, zeros16, ones_w)
    return scat(mof, src2d, zeros64)


def _sc_gather(tab, src2d):
    @functools.partial(
        pl.kernel, mesh=_mesh(),
        out_type=jax.ShapeDtypeStruct((E, 128), F32),
        scratch_types=[pltpu.VMEM((W,), jnp.int32),
                       pltpu.VMEM((W, 128), F32),
                       pltpu.SemaphoreType.DMA],
    )
    def gath(tab_h, src_h, g_h, idx_v, rows_v, sem):
        c = lax.axis_index("c")
        s = lax.axis_index("s")
        wid = s * NC + c

        def body(i, carry):
            widx = wid + NWORK * i

            @pl.when(widx < NWIN)
            def _():
                pltpu.sync_copy(src_h.at[pl.ds(widx * W, W)], idx_v)
                pltpu.async_copy(tab_h.at[idx_v], rows_v, sem).wait()
                pltpu.sync_copy(rows_v, g_h.at[pl.ds(widx * W, W), :])
            return carry

        lax.fori_loop(0, SC_ITERS, body, 0)

    return gath(tab, src2d)


def _dot(a, b):
    return jnp.dot(a, b, preferred_element_type=F32)


def _dott(a, b):
    return lax.dot_general(a, b, (((0,), (0,)), ((), ())),
                           preferred_element_type=F32)


def _onehot(dstv, nrows):
    return (lax.broadcasted_iota(jnp.int32, (nrows, NF), 1) == dstv
            ).astype(F32)


_SMALL = pl.BlockSpec((NF, NF), lambda i: (0, 0))
_ROW = pl.BlockSpec((1, NF), lambda i: (0, 0))
_COL = pl.BlockSpec((NF, 1), lambda i: (0, 0))


def _tc_pass0(ev2, dst2, A0, wmf_e0, t2b, wmo_e0):
    def body(ev_r, dst_r, a0_r, wmfe_r, t2b_r, wmoe_r, mof_r, aggf_r, cnt_r):
        i = pl.program_id(0)
        ev = ev_r[...]
        oh = _onehot(dst_r[...], BE)
        mof_r[...] = jnp.maximum(_dot(oh, a0_r[...]) + ev * wmfe_r[...], 0.0)
        mfo = jnp.maximum(t2b_r[...] + ev * wmoe_r[...], 0.0)
        part = _dott(oh, mfo)
        cpart = _dott(oh, jnp.ones((BE, 1), F32))

        @pl.when(i == 0)
        def _():
            aggf_r[...] = jnp.zeros_like(aggf_r)
            cnt_r[...] = jnp.zeros_like(cnt_r)

        aggf_r[...] += part
        cnt_r[...] += cpart

    return pl.pallas_call(
        body,
        grid=(GE,),
        in_specs=[pl.BlockSpec((BE, 1), lambda i: (i, 0)),
                  pl.BlockSpec((BE, 1), lambda i: (i, 0)),
                  _SMALL, _ROW, _ROW, _ROW],
        out_specs=[pl.BlockSpec((BE, NF), lambda i: (i, 0)),
                   _SMALL, _COL],
        out_shape=[jax.ShapeDtypeStruct((E, NF), F32),
                   jax.ShapeDtypeStruct((NF, NF), F32),
                   jax.ShapeDtypeStruct((NF, 1), F32)],
    )(ev2, dst2, A0, wmf_e0, t2b, wmo_e0)


def _tc_node0(sums, cnts, wn_a0, vn0, we_n0, wmo_n1,
              aggf, cntd, f0b, wf_a0, we_f0, be0, wmf_f1, bmf1):
    def body(s_r, c_r, wna_r, vn0_r, wen_r, wmon_r,
             aggf_r, cntd_r, f0b_r, wfa_r, wef_r, be0_r, wmff_r, bmf1_r,
             nemb1_r, tab_r, inv_r, femb1_r, uf0b_r, tf1b_r):
        i = pl.program_id(0)
        ssum = s_r[0] + s_r[1]
        cnt = jnp.sum(c_r[0] + c_r[1], axis=1, keepdims=True) * (1.0 / 16.0)
        inv = 1.0 / jnp.maximum(cnt, 1.0)
        inv_r[...] = inv
        agg = ssum * inv
        nn0 = jnp.maximum(vn0_r[...] + _dot(agg, wna_r[...]), 0.0)
        nemb1 = jnp.maximum(nn0 + 1.0, 0.0)
        nemb1_r[...] = nemb1
        tab_r[...] = jnp.concatenate(
            [_dot(nn0, wen_r[...]), _dot(nemb1, wmon_r[...])], axis=1)

        @pl.when(i == 0)
        def _():
            invd = 1.0 / jnp.maximum(cntd_r[...], 1.0)
            aggf0 = aggf_r[...] * invd
            nf0 = jnp.maximum(f0b_r[...] + _dot(aggf0, wfa_r[...]), 0.0)
            ey = (lax.broadcasted_iota(jnp.int32, (NF, NF), 0)
                  == lax.broadcasted_iota(jnp.int32, (NF, NF), 1)).astype(F32)
            femb1 = jnp.maximum(nf0 + ey, 0.0)
            femb1_r[...] = femb1
            uf0b_r[...] = _dot(nf0, wef_r[...]) + be0_r[...]
            tf1b_r[...] = _dot(femb1, wmff_r[...]) + bmf1_r[...]

    return pl.pallas_call(
        body,
        grid=(GN,),
        in_specs=[pl.BlockSpec((NC, BN, NF), lambda i: (0, i, 0)),
                  pl.BlockSpec((NC, BN, 16), lambda i: (0, i, 0)),
                  _SMALL, _ROW, _SMALL, _SMALL,
                  _SMALL, _COL, _SMALL, _SMALL, _SMALL, _ROW, _SMALL, _ROW],
        out_specs=[pl.BlockSpec((BN, NF), lambda i: (i, 0)),
                   pl.BlockSpec((BN, 2 * NF), lambda i: (i, 0)),
                   pl.BlockSpec((BN, 1), lambda i: (i, 0)),
                   _SMALL, _SMALL, _SMALL],
        out_shape=[jax.ShapeDtypeStruct((N, NF), F32),
                   jax.ShapeDtypeStruct((N, 2 * NF), F32),
                   jax.ShapeDtypeStruct((N, 1), F32),
                   jax.ShapeDtypeStruct((NF, NF), F32),
                   jax.ShapeDtypeStruct((NF, NF), F32),
                   jax.ShapeDtypeStruct((NF, NF), F32)],
    )(sums, cnts, wn_a0, vn0, we_n0, wmo_n1,
      aggf, cntd, f0b, wf_a0, we_f0, be0, wmf_f1, bmf1)


def _tc_pass1(ev2, dst2, g, uf0b, tf1b, wmf_e1, wmo_e1, bmo1, we_e0):
    def body(ev_r, dst_r, g_r, uf_r, tf_r, wmf_r, wmo_r, bmo_r,
             wee_r, e1_r, mof_r, aggf_r):
        i = pl.program_id(0)
        ev = ev_r[...]
        oh = _onehot(dst_r[...], BE)
        gv = g_r[...]
        inner = ev * wee_r[...] + gv[:, :NF] + _dot(oh, uf_r[...])
        e1 = jnp.maximum(jnp.maximum(inner, 0.0) + ev, 0.0)
        e1_r[...] = e1
        mof_r[...] = jnp.maximum(_dot(oh, tf_r[...]) + _dot(e1, wmf_r[...]),
                                 0.0)
        mfo = jnp.maximum(gv[:, NF:] + _dot(e1, wmo_r[...]) + bmo_r[...], 0.0)
        part = _dott(oh, mfo)

        @pl.when(i == 0)
        def _():
            aggf_r[...] = jnp.zeros_like(aggf_r)

        aggf_r[...] += part

    return pl.pallas_call(
        body,
        grid=(GE,),
        in_specs=[pl.BlockSpec((BE, 1), lambda i: (i, 0)),
                  pl.BlockSpec((BE, 1), lambda i: (i, 0)),
                  pl.BlockSpec((BE, 2 * NF), lambda i: (i, 0)),
                  _SMALL, _SMALL, _SMALL, _SMALL, _ROW, _ROW],
        out_specs=[pl.BlockSpec((BE, NF), lambda i: (i, 0)),
                   pl.BlockSpec((BE, NF), lambda i: (i, 0)),
                   _SMALL],
        out_shape=[jax.ShapeDtypeStruct((E, NF), F32),
                   jax.ShapeDtypeStruct((E, NF), F32),
                   jax.ShapeDtypeStruct((NF, NF), F32)],
    )(ev2, dst2, g, uf0b, tf1b, wmf_e1, wmo_e1, bmo1, we_e0)


def _tc_node1(nemb1, sums, inv, wn_n1, wn_a1, bn1, we_n1, wmo_n2,
              femb1, aggf, cntd, wf_f1, wf_a1, bf1, we_f1, be1, wmf_f2, bmf2):
    def body(ne_r, s_r, inv_r, wnn_r, wna_r, bn_r, wen_r, wmon_r,
             fe_r, aggf_r, cntd_r, wff_r, wfa_r, bf_r, wef_r, be_r,
             wmff_r, bmf_r,
             nemb2_r, tab_r, femb2_r, uf1b_r, tf2b_r):
        i = pl.program_id(0)
        ne = ne_r[...]
        agg = (s_r[0] + s_r[1]) * inv_r[...]
        nn1 = jnp.maximum(_dot(ne, wnn_r[...]) + _dot(agg, wna_r[...])
                          + bn_r[...], 0.0)
        nemb2 = jnp.maximum(nn1 + ne, 0.0)
        nemb2_r[...] = nemb2
        tab_r[...] = jnp.concatenate(
            [_dot(nn1, wen_r[...]), _dot(nemb2, wmon_r[...])], axis=1)

        @pl.when(i == 0)
        def _():
            fe = fe_r[...]
            invd = 1.0 / jnp.maximum(cntd_r[...], 1.0)
            aggf1 = aggf_r[...] * invd
            nf1 = jnp.maximum(_dot(fe, wff_r[...]) + _dot(aggf1, wfa_r[...])
                              + bf_r[...], 0.0)
            femb2 = jnp.maximum(nf1 + fe, 0.0)
            femb2_r[...] = femb2
            uf1b_r[...] = _dot(nf1, wef_r[...]) + be_r[...]
            tf2b_r[...] = _dot(femb2, wmff_r[...]) + bmf_r[...]

    return pl.pallas_call(
        body,
        grid=(GN,),
        in_specs=[pl.BlockSpec((BN, NF), lambda i: (i, 0)),
                  pl.BlockSpec((NC, BN, NF), lambda i: (0, i, 0)),
                  pl.BlockSpec((BN, 1), lambda i: (i, 0)),
                  _SMALL, _SMALL, _ROW, _SMALL, _SMALL,
                  _SMALL, _SMALL, _COL, _SMALL, _SMALL, _ROW, _SMALL, _ROW,
                  _SMALL, _ROW],
        out_specs=[pl.BlockSpec((BN, NF), lambda i: (i, 0)),
                   pl.BlockSpec((BN, 2 * NF), lambda i: (i, 0)),
                   _SMALL, _SMALL, _SMALL],
        out_shape=[jax.ShapeDtypeStruct((N, NF), F32),
                   jax.ShapeDtypeStruct((N, 2 * NF), F32),
                   jax.ShapeDtypeStruct((NF, NF), F32),
                   jax.ShapeDtypeStruct((NF, NF), F32),
                   jax.ShapeDtypeStruct((NF, NF), F32)],
    )(nemb1, sums, inv, wn_n1, wn_a1, bn1, we_n1, wmo_n2,
      femb1, aggf, cntd, wf_f1, wf_a1, bf1, we_f1, be1, wmf_f2, bmf2)


def _tc_pass2(e1, dst2, g, uf1b, tf2b, we_e1, wmf_e2, wmo_e2, bmo2):
    def body(e1_r, dst_r, g_r, uf_r, tf_r, wee_r, wmf_r, wmo_r,
             bmo_r, mof_r, aggf_r):
        i = pl.program_id(0)
        e1v = e1_r[...]
        oh = _onehot(dst_r[...], BE)
        gv = g_r[...]
        inner = _dot(e1v, wee_r[...]) + gv[:, :NF] + _dot(oh, uf_r[...])
        e2 = jnp.maximum(jnp.maximum(inner, 0.0) + e1v, 0.0)
        mof_r[...] = jnp.maximum(_dot(oh, tf_r[...]) + _dot(e2, wmf_r[...]),
                                 0.0)
        mfo = jnp.maximum(gv[:, NF:] + _dot(e2, wmo_r[...]) + bmo_r[...], 0.0)
        part = _dott(oh, mfo)

        @pl.when(i == 0)
        def _():
            aggf_r[...] = jnp.zeros_like(aggf_r)

        aggf_r[...] += part

    return pl.pallas_call(
        body,
        grid=(GE,),
        in_specs=[pl.BlockSpec((BE, NF), lambda i: (i, 0)),
                  pl.BlockSpec((BE, 1), lambda i: (i, 0)),
                  pl.BlockSpec((BE, 2 * NF), lambda i: (i, 0)),
                  _SMALL, _SMALL, _SMALL, _SMALL, _SMALL, _ROW],
        out_specs=[pl.BlockSpec((BE, NF), lambda i: (i, 0)),
                   _SMALL],
        out_shape=[jax.ShapeDtypeStruct((E, NF), F32),
                   jax.ShapeDtypeStruct((NF, NF), F32)],
    )(e1, dst2, g, uf1b, tf2b, we_e1, wmf_e2, wmo_e2, bmo2)


def _tc_node2_head(nemb2, sums, inv, wn_n2, wn_a2, bn2,
                   femb2, aggf, cntd, wf_f2, wf_a2, bf2,
                   wo, wfe, bh, wout, bout, w1, b1, w2, b2):
    def body(ne_r, s_r, inv_r, wnn_r, wna_r, bn_r,
             fe_r, aggf_r, cntd_r, wff_r, wfa_r, bf_r,
             wo_r, wfe_r, bh_r, wout_r, bout_r, w1_r, b1_r, w2_r, b2_r,
             dhat_r, yhat_r):
        ne = ne_r[...]
        agg = (s_r[0] + s_r[1]) * inv_r[...]
        nn2 = jnp.maximum(_dot(ne, wnn_r[...]) + _dot(agg, wna_r[...])
                          + bn_r[...], 0.0)
        nemb3 = jnp.maximum(nn2 + ne, 0.0)
        obs_h = _dot(nemb3, wo_r[...])
        fe = fe_r[...]
        invd = 1.0 / jnp.maximum(cntd_r[...], 1.0)
        aggf2 = aggf_r[...] * invd
        nf2 = jnp.maximum(_dot(fe, wff_r[...]) + _dot(aggf2, wfa_r[...])
                          + bf_r[...], 0.0)
        femb3 = jnp.maximum(nf2 + fe, 0.0)
        cmat = _dot(femb3, wfe_r[...]) + bh_r[...]
        h3 = jnp.maximum(obs_h[:, None, :] + cmat[None, :, :], 0.0)
        d = (_dot(h3.reshape(BH * NF, NF), wout_r[...]).reshape(BH, NF)
             + bout_r[...])
        dhat_r[...] = d
        y = (_dot(jnp.maximum(_dot(d, w1_r[...]) + b1_r[...], 0.0), w2_r[...])
             + b2_r[...])
        yhat_r[...] = y

    return pl.pallas_call(
        body,
        grid=(GH,),
        in_specs=[pl.BlockSpec((BH, NF), lambda i: (i, 0)),
                  pl.BlockSpec((NC, BH, NF), lambda i: (0, i, 0)),
                  pl.BlockSpec((BH, 1), lambda i: (i, 0)),
                  _SMALL, _SMALL, _ROW,
                  _SMALL, _SMALL, _COL, _SMALL, _SMALL, _ROW,
                  _SMALL, _SMALL, _ROW, _COL, pl.BlockSpec((1, 1), lambda i: (0, 0)),
                  _SMALL, _ROW, _COL, pl.BlockSpec((1, 1), lambda i: (0, 0))],
        out_specs=[pl.BlockSpec((BH, NF), lambda i: (i, 0)),
                   pl.BlockSpec((BH, 1), lambda i: (i, 0))],
        out_shape=[jax.ShapeDtypeStruct((N, NF), F32),
                   jax.ShapeDtypeStruct((N, 1), F32)],
    )(nemb2, sums, inv, wn_n2, wn_a2, bn2,
      femb2, aggf, cntd, wf_f2, wf_a2, bf2,
      wo, wfe, bh, wout, bout, w1, b1, w2, b2)


def _sc_scatter_jnp(mof, src2d, zeros64, zeros16, ones_w, with_counts):
    src = src2d.reshape(-1)
    s = jax.ops.segment_sum(mof, src, num_segments=N)
    sums = jnp.stack([s, jnp.zeros_like(s)])
    if with_counts:
        c = jax.ops.segment_sum(jnp.ones((E, 16), F32), src, num_segments=N)
        return sums, jnp.stack([c, jnp.zeros_like(c)])
    return sums


def kernel(x, edge_index, edge_value, params):
    del x
    src = edge_index[0]
    dst = edge_index[1]
    ev2 = edge_value.reshape(E, 1)
    dst2 = dst.reshape(E, 1)
    src2d = src
    zeros64 = jnp.zeros((N, 64), F32)
    zeros16 = jnp.zeros((N, 16), F32)
    ones_w = jnp.ones((W, 16), F32)

    p0, p1, p2 = params["block0"], params["block1"], params["block2"]
    eph, nph = params["eph"], params["nph"]

    def row(v):
        return v.reshape(1, NF)

    A0 = p0["Wmf"][:64] + row(p0["bmf"])
    wmf_e0 = p0["Wmf"][64:65]
    t2b = jnp.sum(p0["Wmo"][:64], 0, keepdims=True) + row(p0["bmo"])
    wmo_e0 = p0["Wmo"][64:65]
    vn0 = jnp.sum(p0["Wn"][:64], 0, keepdims=True) + row(p0["bn"])
    wn_a0 = p0["Wn"][64:]
    f0b = p0["Wf"][:64] + row(p0["bf"])
    wf_a0 = p0["Wf"][64:]
    we_e0 = p0["We"][0:1]
    we_n0 = p0["We"][1:65]
    we_f0 = p0["We"][65:]
    be0 = row(p0["be"])

    wmf_f1, wmf_e1 = p1["Wmf"][:64], p1["Wmf"][64:]
    wmo_n1, wmo_e1 = p1["Wmo"][:64], p1["Wmo"][64:]
    wn_n1, wn_a1 = p1["Wn"][:64], p1["Wn"][64:]
    wf_f1, wf_a1 = p1["Wf"][:64], p1["Wf"][64:]
    we_e1, we_n1, we_f1 = p1["We"][:64], p1["We"][64:128], p1["We"][128:]
    bmf1, bmo1, bn1, bf1, be1 = (row(p1["bmf"]), row(p1["bmo"]),
                                 row(p1["bn"]), row(p1["bf"]), row(p1["be"]))

    wmf_f2, wmf_e2 = p2["Wmf"][:64], p2["Wmf"][64:]
    wmo_n2, wmo_e2 = p2["Wmo"][:64], p2["Wmo"][64:]
    wn_n2, wn_a2 = p2["Wn"][:64], p2["Wn"][64:]
    wf_f2, wf_a2 = p2["Wf"][:64], p2["Wf"][64:]
    bmf2, bmo2, bn2, bf2 = (row(p2["bmf"]), row(p2["bmo"]),
                            row(p2["bn"]), row(p2["bf"]))

    mof0, aggf0, cntd = _tc_pass0(ev2, dst2, A0, wmf_e0, t2b, wmo_e0)
    sums0, cnts = _sc_scatter_jnp(mof0, src2d, zeros64, zeros16, ones_w, True)

    (nemb1, tab1, inv, femb1, uf0b, tf1b) = _tc_node0(
        sums0, cnts, wn_a0, vn0, we_n0, wmo_n1,
        aggf0, cntd, f0b, wf_a0, we_f0, be0, wmf_f1, bmf1)

    g0 = _sc_gather(tab1, src2d)
    e1, mof1, aggf1 = _tc_pass1(ev2, dst2, g0, uf0b, tf1b,
                                wmf_e1, wmo_e1, bmo1, we_e0)
    sums1 = _sc_scatter_jnp(mof1, src2d, zeros64, zeros16, ones_w, False)

    (nemb2, tab2, femb2, uf1b, tf2b) = _tc_node1(
        nemb1, sums1, inv, wn_n1, wn_a1, bn1, we_n1, wmo_n2,
        femb1, aggf1, cntd, wf_f1, wf_a1, bf1, we_f1, be1, wmf_f2, bmf2)

    g1 = _sc_gather(tab2, src2d)
    mof2, aggf2 = _tc_pass2(e1, dst2, g1, uf1b, tf2b,
                            we_e1, wmf_e2, wmo_e2, bmo2)
    sums2 = _sc_scatter_jnp(mof2, src2d, zeros64, zeros16, ones_w, False)

    d_hat, y_hat = _tc_node2_head(
        nemb2, sums2, inv, wn_n2, wn_a2, bn2,
        femb2, aggf2, cntd, wf_f2, wf_a2, bf2,
        eph["Wo"], eph["Wf"], row(eph["bh"]), eph["wout"].reshape(NF, 1),
        eph["bout"].reshape(1, 1), nph["W1"], row(nph["b1"]),
        nph["W2"], nph["b2"].reshape(1, 1))
    return d_hat, y_hat


# speedup vs baseline: 4.3667x; 1.7315x over previous
import functools

import jax
import jax.numpy as jnp
from jax import lax
from jax.experimental import pallas as pl
from jax.experimental.pallas import tpu as pltpu
from jax.experimental.pallas import tpu_sc as plsc

F32 = jnp.float32

E = 320000
N = 10000
NF = 64
W = 128
NWIN = E // W
NC, NS = 2, 16
NWORK = NC * NS
CH = 624
SC_ITERS = (NWIN + NWORK - 1) // NWORK

BE = 2000
GE = E // BE
BN = 2000
GN = N // BN
BH = 400
GH = N // BH

def _mesh():
    return plsc.VectorSubcoreMesh(core_axis_name="c", subcore_axis_name="s")


def _sc_scatter(mof, src2d, zeros128):
    @functools.partial(
        pl.kernel, mesh=_mesh(),
        out_type=jax.ShapeDtypeStruct((NC, N, 128), F32),
        scratch_types=[pltpu.VMEM((W,), jnp.int32),
                       pltpu.VMEM((W, 128), F32),
                       pltpu.VMEM_SHARED((N, 128), F32)])
    def scat(mof_h, src_h, z_h, out_h, idx_v, rows_v, acc):
        c = lax.axis_index("c")
        s = lax.axis_index("s")
        wid = s * NC + c
        r0 = s * CH

        pltpu.sync_copy(z_h.at[pl.ds(r0, CH), :], acc.at[pl.ds(r0, CH), :])

        @pl.when(s == NS - 1)
        def _():
            pltpu.sync_copy(z_h.at[pl.ds(N - 16, 16), :],
                            acc.at[pl.ds(N - 16, 16), :])

        plsc.subcore_barrier()

        def body(i, carry):
            widx = wid + NWORK * i

            @pl.when(widx < NWIN)
            def _():
                pltpu.sync_copy(src_h.at[pl.ds(widx * W, W)], idx_v)
                pltpu.sync_copy(mof_h.at[pl.ds(widx * W, W), :], rows_v)
                pltpu.sync_copy(rows_v, acc.at[idx_v], add=True)
            return carry

        lax.fori_loop(0, SC_ITERS, body, 0)
        plsc.subcore_barrier()
        pltpu.sync_copy(acc.at[pl.ds(r0, CH), :], out_h.at[c, pl.ds(r0, CH), :])

        @pl.when(s == NS - 1)
        def _():
            pltpu.sync_copy(acc.at[pl.ds(N - 16, 16), :],
                            out_h.at[c, pl.ds(N - 16, 16), :])

    return scat(mof, src2d, zeros128)


def _sc_gather(tab, src2d):
    @functools.partial(
        pl.kernel, mesh=_mesh(),
        out_type=jax.ShapeDtypeStruct((E, 128), F32),
        scratch_types=[pltpu.VMEM((W,), jnp.int32),
                       pltpu.VMEM((W, 128), F32),
                       pltpu.SemaphoreType.DMA],
    )
    def gath(tab_h, src_h, g_h, idx_v, rows_v, sem):
        c = lax.axis_index("c")
        s = lax.axis_index("s")
        wid = s * NC + c

        def body(i, carry):
            widx = wid + NWORK * i

            @pl.when(widx < NWIN)
            def _():
                pltpu.sync_copy(src_h.at[pl.ds(widx * W, W)], idx_v)
                pltpu.async_copy(tab_h.at[idx_v], rows_v, sem).wait()
                pltpu.sync_copy(rows_v, g_h.at[pl.ds(widx * W, W), :])
            return carry

        lax.fori_loop(0, SC_ITERS, body, 0)

    return gath(tab, src2d)


def _dot(a, b):
    return jnp.dot(a, b, preferred_element_type=F32)


def _dott(a, b):
    return lax.dot_general(a, b, (((0,), (0,)), ((), ())),
                           preferred_element_type=F32)


def _onehot(dstv, nrows):
    return (lax.broadcasted_iota(jnp.int32, (nrows, NF), 1) == dstv
            ).astype(F32)


_SMALL = pl.BlockSpec((NF, NF), lambda i: (0, 0))
_ROW = pl.BlockSpec((1, NF), lambda i: (0, 0))
_COL = pl.BlockSpec((NF, 1), lambda i: (0, 0))


def _payload(m):
    return jnp.concatenate(
        [m, jnp.ones((m.shape[0], 1), F32), jnp.zeros((m.shape[0], 63), F32)],
        axis=1)


def _tc_pass0(ev2, dst2, A0, wmf_e0, t2b, wmo_e0):
    def body(ev_r, dst_r, a0_r, wmfe_r, t2b_r, wmoe_r, mof_r, aggf_r, cnt_r):
        i = pl.program_id(0)
        ev = ev_r[...]
        oh = _onehot(dst_r[...], BE)
        mof_r[...] = _payload(
            jnp.maximum(_dot(oh, a0_r[...]) + ev * wmfe_r[...], 0.0))
        mfo = jnp.maximum(t2b_r[...] + ev * wmoe_r[...], 0.0)
        part = _dott(oh, mfo)
        cpart = _dott(oh, jnp.ones((BE, 1), F32))

        @pl.when(i == 0)
        def _():
            aggf_r[...] = jnp.zeros_like(aggf_r)
            cnt_r[...] = jnp.zeros_like(cnt_r)

        aggf_r[...] += part
        cnt_r[...] += cpart

    return pl.pallas_call(
        body,
        grid=(GE,),
        in_specs=[pl.BlockSpec((BE, 1), lambda i: (i, 0)),
                  pl.BlockSpec((BE, 1), lambda i: (i, 0)),
                  _SMALL, _ROW, _ROW, _ROW],
        out_specs=[pl.BlockSpec((BE, 2 * NF), lambda i: (i, 0)),
                   _SMALL, _COL],
        out_shape=[jax.ShapeDtypeStruct((E, 2 * NF), F32),
                   jax.ShapeDtypeStruct((NF, NF), F32),
                   jax.ShapeDtypeStruct((NF, 1), F32)],
    )(ev2, dst2, A0, wmf_e0, t2b, wmo_e0)


def _tc_node0(sums, wn_a0, vn0, we_n0, wmo_n1,
              aggf, cntd, f0b, wf_a0, we_f0, be0, wmf_f1, bmf1):
    def body(s_r, wna_r, vn0_r, wen_r, wmon_r,
             aggf_r, cntd_r, f0b_r, wfa_r, wef_r, be0_r, wmff_r, bmf1_r,
             nemb1_r, tab_r, inv_r, femb1_r, uf0b_r, tf1b_r):
        i = pl.program_id(0)
        sfull = s_r[0] + s_r[1]
        ssum = sfull[:, :NF]
        cnt = jnp.sum(sfull[:, NF:], axis=1, keepdims=True)
        inv = 1.0 / jnp.maximum(cnt, 1.0)
        inv_r[...] = inv
        agg = ssum * inv
        nn0 = jnp.maximum(vn0_r[...] + _dot(agg, wna_r[...]), 0.0)
        nemb1 = jnp.maximum(nn0 + 1.0, 0.0)
        nemb1_r[...] = nemb1
        tab_r[...] = jnp.concatenate(
            [_dot(nn0, wen_r[...]), _dot(nemb1, wmon_r[...])], axis=1)

        @pl.when(i == 0)
        def _():
            invd = 1.0 / jnp.maximum(cntd_r[...], 1.0)
            aggf0 = aggf_r[...] * invd
            nf0 = jnp.maximum(f0b_r[...] + _dot(aggf0, wfa_r[...]), 0.0)
            ey = (lax.broadcasted_iota(jnp.int32, (NF, NF), 0)
                  == lax.broadcasted_iota(jnp.int32, (NF, NF), 1)).astype(F32)
            femb1 = jnp.maximum(nf0 + ey, 0.0)
            femb1_r[...] = femb1
            uf0b_r[...] = _dot(nf0, wef_r[...]) + be0_r[...]
            tf1b_r[...] = _dot(femb1, wmff_r[...]) + bmf1_r[...]

    return pl.pallas_call(
        body,
        grid=(GN,),
        in_specs=[pl.BlockSpec((NC, BN, 2 * NF), lambda i: (0, i, 0)),
                  _SMALL, _ROW, _SMALL, _SMALL,
                  _SMALL, _COL, _SMALL, _SMALL, _SMALL, _ROW, _SMALL, _ROW],
        out_specs=[pl.BlockSpec((BN, NF), lambda i: (i, 0)),
                   pl.BlockSpec((BN, 2 * NF), lambda i: (i, 0)),
                   pl.BlockSpec((BN, 1), lambda i: (i, 0)),
                   _SMALL, _SMALL, _SMALL],
        out_shape=[jax.ShapeDtypeStruct((N, NF), F32),
                   jax.ShapeDtypeStruct((N, 2 * NF), F32),
                   jax.ShapeDtypeStruct((N, 1), F32),
                   jax.ShapeDtypeStruct((NF, NF), F32),
                   jax.ShapeDtypeStruct((NF, NF), F32),
                   jax.ShapeDtypeStruct((NF, NF), F32)],
    )(sums, wn_a0, vn0, we_n0, wmo_n1,
      aggf, cntd, f0b, wf_a0, we_f0, be0, wmf_f1, bmf1)


def _tc_pass1(ev2, dst2, g, uf0b, tf1b, wmf_e1, wmo_e1, bmo1, we_e0):
    def body(ev_r, dst_r, g_r, uf_r, tf_r, wmf_r, wmo_r, bmo_r,
             wee_r, e1_r, mof_r, aggf_r):
        i = pl.program_id(0)
        ev = ev_r[...]
        oh = _onehot(dst_r[...], BE)
        gv = g_r[...]
        inner = ev * wee_r[...] + gv[:, :NF] + _dot(oh, uf_r[...])
        e1 = jnp.maximum(jnp.maximum(inner, 0.0) + ev, 0.0)
        e1_r[...] = e1
        mof_r[...] = _payload(
            jnp.maximum(_dot(oh, tf_r[...]) + _dot(e1, wmf_r[...]), 0.0))
        mfo = jnp.maximum(gv[:, NF:] + _dot(e1, wmo_r[...]) + bmo_r[...], 0.0)
        part = _dott(oh, mfo)

        @pl.when(i == 0)
        def _():
            aggf_r[...] = jnp.zeros_like(aggf_r)

        aggf_r[...] += part

    return pl.pallas_call(
        body,
        grid=(GE,),
        in_specs=[pl.BlockSpec((BE, 1), lambda i: (i, 0)),
                  pl.BlockSpec((BE, 1), lambda i: (i, 0)),
                  pl.BlockSpec((BE, 2 * NF), lambda i: (i, 0)),
                  _SMALL, _SMALL, _SMALL, _SMALL, _ROW, _ROW],
        out_specs=[pl.BlockSpec((BE, NF), lambda i: (i, 0)),
                   pl.BlockSpec((BE, 2 * NF), lambda i: (i, 0)),
                   _SMALL],
        out_shape=[jax.ShapeDtypeStruct((E, NF), F32),
                   jax.ShapeDtypeStruct((E, 2 * NF), F32),
                   jax.ShapeDtypeStruct((NF, NF), F32)],
    )(ev2, dst2, g, uf0b, tf1b, wmf_e1, wmo_e1, bmo1, we_e0)


def _tc_node1(nemb1, sums, inv, wn_n1, wn_a1, bn1, we_n1, wmo_n2,
              femb1, aggf, cntd, wf_f1, wf_a1, bf1, we_f1, be1, wmf_f2, bmf2):
    def body(ne_r, s_r, inv_r, wnn_r, wna_r, bn_r, wen_r, wmon_r,
             fe_r, aggf_r, cntd_r, wff_r, wfa_r, bf_r, wef_r, be_r,
             wmff_r, bmf_r,
             nemb2_r, tab_r, femb2_r, uf1b_r, tf2b_r):
        i = pl.program_id(0)
        ne = ne_r[...]
        agg = (s_r[0] + s_r[1])[:, :NF] * inv_r[...]
        nn1 = jnp.maximum(_dot(ne, wnn_r[...]) + _dot(agg, wna_r[...])
                          + bn_r[...], 0.0)
        nemb2 = jnp.maximum(nn1 + ne, 0.0)
        nemb2_r[...] = nemb2
        tab_r[...] = jnp.concatenate(
            [_dot(nn1, wen_r[...]), _dot(nemb2, wmon_r[...])], axis=1)

        @pl.when(i == 0)
        def _():
            fe = fe_r[...]
            invd = 1.0 / jnp.maximum(cntd_r[...], 1.0)
            aggf1 = aggf_r[...] * invd
            nf1 = jnp.maximum(_dot(fe, wff_r[...]) + _dot(aggf1, wfa_r[...])
                              + bf_r[...], 0.0)
            femb2 = jnp.maximum(nf1 + fe, 0.0)
            femb2_r[...] = femb2
            uf1b_r[...] = _dot(nf1, wef_r[...]) + be_r[...]
            tf2b_r[...] = _dot(femb2, wmff_r[...]) + bmf_r[...]

    return pl.pallas_call(
        body,
        grid=(GN,),
        in_specs=[pl.BlockSpec((BN, NF), lambda i: (i, 0)),
                  pl.BlockSpec((NC, BN, 2 * NF), lambda i: (0, i, 0)),
                  pl.BlockSpec((BN, 1), lambda i: (i, 0)),
                  _SMALL, _SMALL, _ROW, _SMALL, _SMALL,
                  _SMALL, _SMALL, _COL, _SMALL, _SMALL, _ROW, _SMALL, _ROW,
                  _SMALL, _ROW],
        out_specs=[pl.BlockSpec((BN, NF), lambda i: (i, 0)),
                   pl.BlockSpec((BN, 2 * NF), lambda i: (i, 0)),
                   _SMALL, _SMALL, _SMALL],
        out_shape=[jax.ShapeDtypeStruct((N, NF), F32),
                   jax.ShapeDtypeStruct((N, 2 * NF), F32),
                   jax.ShapeDtypeStruct((NF, NF), F32),
                   jax.ShapeDtypeStruct((NF, NF), F32),
                   jax.ShapeDtypeStruct((NF, NF), F32)],
    )(nemb1, sums, inv, wn_n1, wn_a1, bn1, we_n1, wmo_n2,
      femb1, aggf, cntd, wf_f1, wf_a1, bf1, we_f1, be1, wmf_f2, bmf2)


def _tc_pass2(e1, dst2, g, uf1b, tf2b, we_e1, wmf_e2, wmo_e2, bmo2):
    def body(e1_r, dst_r, g_r, uf_r, tf_r, wee_r, wmf_r, wmo_r,
             bmo_r, mof_r, aggf_r):
        i = pl.program_id(0)
        e1v = e1_r[...]
        oh = _onehot(dst_r[...], BE)
        gv = g_r[...]
        inner = _dot(e1v, wee_r[...]) + gv[:, :NF] + _dot(oh, uf_r[...])
        e2 = jnp.maximum(jnp.maximum(inner, 0.0) + e1v, 0.0)
        mof_r[...] = _payload(
            jnp.maximum(_dot(oh, tf_r[...]) + _dot(e2, wmf_r[...]), 0.0))
        mfo = jnp.maximum(gv[:, NF:] + _dot(e2, wmo_r[...]) + bmo_r[...], 0.0)
        part = _dott(oh, mfo)

        @pl.when(i == 0)
        def _():
            aggf_r[...] = jnp.zeros_like(aggf_r)

        aggf_r[...] += part

    return pl.pallas_call(
        body,
        grid=(GE,),
        in_specs=[pl.BlockSpec((BE, NF), lambda i: (i, 0)),
                  pl.BlockSpec((BE, 1), lambda i: (i, 0)),
                  pl.BlockSpec((BE, 2 * NF), lambda i: (i, 0)),
                  _SMALL, _SMALL, _SMALL, _SMALL, _SMALL, _ROW],
        out_specs=[pl.BlockSpec((BE, 2 * NF), lambda i: (i, 0)),
                   _SMALL],
        out_shape=[jax.ShapeDtypeStruct((E, 2 * NF), F32),
                   jax.ShapeDtypeStruct((NF, NF), F32)],
    )(e1, dst2, g, uf1b, tf2b, we_e1, wmf_e2, wmo_e2, bmo2)


def _tc_node2_head(nemb2, sums, inv, wn_n2, wn_a2, bn2,
                   femb2, aggf, cntd, wf_f2, wf_a2, bf2,
                   wo, wfe, bh, wout, bout, w1, b1, w2, b2):
    def body(ne_r, s_r, inv_r, wnn_r, wna_r, bn_r,
             fe_r, aggf_r, cntd_r, wff_r, wfa_r, bf_r,
             wo_r, wfe_r, bh_r, wout_r, bout_r, w1_r, b1_r, w2_r, b2_r,
             dhat_r, yhat_r):
        ne = ne_r[...]
        agg = (s_r[0] + s_r[1])[:, :NF] * inv_r[...]
        nn2 = jnp.maximum(_dot(ne, wnn_r[...]) + _dot(agg, wna_r[...])
                          + bn_r[...], 0.0)
        nemb3 = jnp.maximum(nn2 + ne, 0.0)
        obs_h = _dot(nemb3, wo_r[...])
        fe = fe_r[...]
        invd = 1.0 / jnp.maximum(cntd_r[...], 1.0)
        aggf2 = aggf_r[...] * invd
        nf2 = jnp.maximum(_dot(fe, wff_r[...]) + _dot(aggf2, wfa_r[...])
                          + bf_r[...], 0.0)
        femb3 = jnp.maximum(nf2 + fe, 0.0)
        cmat = _dot(femb3, wfe_r[...]) + bh_r[...]
        h3 = jnp.maximum(obs_h[:, None, :] + cmat[None, :, :], 0.0)
        d = (_dot(h3.reshape(BH * NF, NF), wout_r[...]).reshape(BH, NF)
             + bout_r[...])
        dhat_r[...] = d
        y = (_dot(jnp.maximum(_dot(d, w1_r[...]) + b1_r[...], 0.0), w2_r[...])
             + b2_r[...])
        yhat_r[...] = y

    return pl.pallas_call(
        body,
        grid=(GH,),
        in_specs=[pl.BlockSpec((BH, NF), lambda i: (i, 0)),
                  pl.BlockSpec((NC, BH, 2 * NF), lambda i: (0, i, 0)),
                  pl.BlockSpec((BH, 1), lambda i: (i, 0)),
                  _SMALL, _SMALL, _ROW,
                  _SMALL, _SMALL, _COL, _SMALL, _SMALL, _ROW,
                  _SMALL, _SMALL, _ROW, _COL, pl.BlockSpec((1, 1), lambda i: (0, 0)),
                  _SMALL, _ROW, _COL, pl.BlockSpec((1, 1), lambda i: (0, 0))],
        out_specs=[pl.BlockSpec((BH, NF), lambda i: (i, 0)),
                   pl.BlockSpec((BH, 1), lambda i: (i, 0))],
        out_shape=[jax.ShapeDtypeStruct((N, NF), F32),
                   jax.ShapeDtypeStruct((N, 1), F32)],
    )(nemb2, sums, inv, wn_n2, wn_a2, bn2,
      femb2, aggf, cntd, wf_f2, wf_a2, bf2,
      wo, wfe, bh, wout, bout, w1, b1, w2, b2)


def kernel(x, edge_index, edge_value, params):
    del x
    src = edge_index[0]
    dst = edge_index[1]
    ev2 = edge_value.reshape(E, 1)
    dst2 = dst.reshape(E, 1)
    src2d = src
    zeros128 = jnp.zeros((N, 128), F32)

    p0, p1, p2 = params["block0"], params["block1"], params["block2"]
    eph, nph = params["eph"], params["nph"]

    def row(v):
        return v.reshape(1, NF)

    A0 = p0["Wmf"][:64] + row(p0["bmf"])
    wmf_e0 = p0["Wmf"][64:65]
    t2b = jnp.sum(p0["Wmo"][:64], 0, keepdims=True) + row(p0["bmo"])
    wmo_e0 = p0["Wmo"][64:65]
    vn0 = jnp.sum(p0["Wn"][:64], 0, keepdims=True) + row(p0["bn"])
    wn_a0 = p0["Wn"][64:]
    f0b = p0["Wf"][:64] + row(p0["bf"])
    wf_a0 = p0["Wf"][64:]
    we_e0 = p0["We"][0:1]
    we_n0 = p0["We"][1:65]
    we_f0 = p0["We"][65:]
    be0 = row(p0["be"])

    wmf_f1, wmf_e1 = p1["Wmf"][:64], p1["Wmf"][64:]
    wmo_n1, wmo_e1 = p1["Wmo"][:64], p1["Wmo"][64:]
    wn_n1, wn_a1 = p1["Wn"][:64], p1["Wn"][64:]
    wf_f1, wf_a1 = p1["Wf"][:64], p1["Wf"][64:]
    we_e1, we_n1, we_f1 = p1["We"][:64], p1["We"][64:128], p1["We"][128:]
    bmf1, bmo1, bn1, bf1, be1 = (row(p1["bmf"]), row(p1["bmo"]),
                                 row(p1["bn"]), row(p1["bf"]), row(p1["be"]))

    wmf_f2, wmf_e2 = p2["Wmf"][:64], p2["Wmf"][64:]
    wmo_n2, wmo_e2 = p2["Wmo"][:64], p2["Wmo"][64:]
    wn_n2, wn_a2 = p2["Wn"][:64], p2["Wn"][64:]
    wf_f2, wf_a2 = p2["Wf"][:64], p2["Wf"][64:]
    bmf2, bmo2, bn2, bf2 = (row(p2["bmf"]), row(p2["bmo"]),
                            row(p2["bn"]), row(p2["bf"]))

    mof0, aggf0, cntd = _tc_pass0(ev2, dst2, A0, wmf_e0, t2b, wmo_e0)
    sums0 = _sc_scatter(mof0, src2d, zeros128)

    (nemb1, tab1, inv, femb1, uf0b, tf1b) = _tc_node0(
        sums0, wn_a0, vn0, we_n0, wmo_n1,
        aggf0, cntd, f0b, wf_a0, we_f0, be0, wmf_f1, bmf1)

    g0 = _sc_gather(tab1, src2d)
    e1, mof1, aggf1 = _tc_pass1(ev2, dst2, g0, uf0b, tf1b,
                                wmf_e1, wmo_e1, bmo1, we_e0)
    sums1 = _sc_scatter(mof1, src2d, zeros128)

    (nemb2, tab2, femb2, uf1b, tf2b) = _tc_node1(
        nemb1, sums1, inv, wn_n1, wn_a1, bn1, we_n1, wmo_n2,
        femb1, aggf1, cntd, wf_f1, wf_a1, bf1, we_f1, be1, wmf_f2, bmf2)

    g1 = _sc_gather(tab2, src2d)
    mof2, aggf2 = _tc_pass2(e1, dst2, g1, uf1b, tf2b,
                            we_e1, wmf_e2, wmo_e2, bmo2)
    sums2 = _sc_scatter(mof2, src2d, zeros128)

    d_hat, y_hat = _tc_node2_head(
        nemb2, sums2, inv, wn_n2, wn_a2, bn2,
        femb2, aggf2, cntd, wf_f2, wf_a2, bf2,
        eph["Wo"], eph["Wf"], row(eph["bh"]), eph["wout"].reshape(NF, 1),
        eph["bout"].reshape(1, 1), nph["W1"], row(nph["b1"]),
        nph["W2"], nph["b2"].reshape(1, 1))
    return d_hat, y_hat


# speedup vs baseline: 5.3981x; 1.2362x over previous
import functools

import jax
import jax.numpy as jnp
from jax import lax
from jax.experimental import pallas as pl
from jax.experimental.pallas import tpu as pltpu
from jax.experimental.pallas import tpu_sc as plsc

F32 = jnp.float32

E = 320000
N = 10000
NF = 64
W = 128
NWIN = E // W
NC, NS = 2, 16
NWORK = NC * NS
CH = 624
WQ, WR = divmod(NWIN, NWORK)

BE = 2000
GE = E // BE
BN = 2000
GN = N // BN
BH = 400
GH = N // BH

def _mesh():
    return plsc.VectorSubcoreMesh(core_axis_name="c", subcore_axis_name="s")


def _worker_range(wid):
    start = wid * WQ + jnp.minimum(wid, WR)
    cnt = jnp.where(wid < WR, WQ + 1, WQ)
    return start, cnt


def _sc_scatter(mof, src2d, zeros128):
    @functools.partial(
        pl.kernel, mesh=_mesh(),
        out_type=jax.ShapeDtypeStruct((NC, N, 128), F32),
        scratch_types=[pltpu.VMEM((W,), jnp.int32),
                       pltpu.VMEM((W,), jnp.int32),
                       pltpu.VMEM((W, 128), F32),
                       pltpu.VMEM((W, 128), F32),
                       pltpu.VMEM_SHARED((N, 128), F32),
                       pltpu.SemaphoreType.DMA,
                       pltpu.SemaphoreType.DMA,
                       pltpu.SemaphoreType.DMA,
                       pltpu.SemaphoreType.DMA])
    def scat(mof_h, src_h, z_h, out_h, idx_a, idx_b, rows_a, rows_b,
             acc, sia, sib, sra, srb):
        c = lax.axis_index("c")
        s = lax.axis_index("s")
        wid = s * NC + c
        r0 = s * CH
        start, cntw = _worker_range(wid)

        def load(widx, idx_v, rows_v, si, sr):
            pltpu.async_copy(src_h.at[pl.ds(widx * W, W)], idx_v, si)
            pltpu.async_copy(mof_h.at[pl.ds(widx * W, W), :], rows_v, sr)

        def drain(idx_v, rows_v, si, sr):
            pltpu.make_async_copy(src_h.at[pl.ds(0, W)], idx_v, si).wait()
            pltpu.make_async_copy(mof_h.at[pl.ds(0, W), :], rows_v, sr).wait()

        load(start, idx_a, rows_a, sia, sra)
        load(start + 1, idx_b, rows_b, sib, srb)
        pltpu.sync_copy(z_h.at[pl.ds(r0, CH), :], acc.at[pl.ds(r0, CH), :])

        @pl.when(s == NS - 1)
        def _():
            pltpu.sync_copy(z_h.at[pl.ds(N - 16, 16), :],
                            acc.at[pl.ds(N - 16, 16), :])

        plsc.subcore_barrier()

        def body(i, carry):
            def step(idx_v, rows_v, si, sr):
                drain(idx_v, rows_v, si, sr)
                pltpu.sync_copy(rows_v, acc.at[idx_v], add=True)

                @pl.when(i + 2 < cntw)
                def _():
                    load(start + i + 2, idx_v, rows_v, si, sr)

            @pl.when(lax.rem(i, 2) == 0)
            def _():
                step(idx_a, rows_a, sia, sra)

            @pl.when(lax.rem(i, 2) == 1)
            def _():
                step(idx_b, rows_b, sib, srb)
            return carry

        lax.fori_loop(0, cntw, body, 0)
        plsc.subcore_barrier()
        pltpu.sync_copy(acc.at[pl.ds(r0, CH), :], out_h.at[c, pl.ds(r0, CH), :])

        @pl.when(s == NS - 1)
        def _():
            pltpu.sync_copy(acc.at[pl.ds(N - 16, 16), :],
                            out_h.at[c, pl.ds(N - 16, 16), :])

    return scat(mof, src2d, zeros128)


def _sc_gather(tab, src2d):
    @functools.partial(
        pl.kernel, mesh=_mesh(),
        out_type=jax.ShapeDtypeStruct((E, 128), F32),
        scratch_types=[pltpu.VMEM((W,), jnp.int32),
                       pltpu.VMEM((W,), jnp.int32),
                       pltpu.VMEM((W, 128), F32),
                       pltpu.VMEM((W, 128), F32),
                       pltpu.SemaphoreType.DMA,
                       pltpu.SemaphoreType.DMA,
                       pltpu.SemaphoreType.DMA,
                       pltpu.SemaphoreType.DMA,
                       pltpu.SemaphoreType.DMA],
    )
    def gath(tab_h, src_h, g_h, idx_a, idx_b, rows_a, rows_b,
             sia, sib, sta, stb, sg):
        c = lax.axis_index("c")
        s = lax.axis_index("s")
        wid = s * NC + c
        start, cntw = _worker_range(wid)

        pltpu.async_copy(src_h.at[pl.ds(start * W, W)], idx_a, sia)
        pltpu.async_copy(src_h.at[pl.ds((start + 1) * W, W)], idx_b, sib)

        def body(i, carry):
            widx = start + i

            def step(idx_v, rows_v, si, st):
                @pl.when(i >= 2)
                def _():
                    pltpu.make_async_copy(
                        rows_v, g_h.at[pl.ds(0, W), :], st).wait()

                pltpu.make_async_copy(
                    src_h.at[pl.ds(0, W)], idx_v, si).wait()
                pltpu.async_copy(tab_h.at[idx_v], rows_v, sg).wait()
                pltpu.async_copy(rows_v, g_h.at[pl.ds(widx * W, W), :], st)

                @pl.when(i + 2 < cntw)
                def _():
                    pltpu.async_copy(
                        src_h.at[pl.ds((widx + 2) * W, W)], idx_v, si)

            @pl.when(lax.rem(i, 2) == 0)
            def _():
                step(idx_a, rows_a, sia, sta)

            @pl.when(lax.rem(i, 2) == 1)
            def _():
                step(idx_b, rows_b, sib, stb)
            return carry

        lax.fori_loop(0, cntw, body, 0)
        pltpu.make_async_copy(rows_a, g_h.at[pl.ds(0, W), :], sta).wait()
        pltpu.make_async_copy(rows_b, g_h.at[pl.ds(0, W), :], stb).wait()

    return gath(tab, src2d)


def _dot(a, b):
    return jnp.dot(a, b, preferred_element_type=F32)


def _dott(a, b):
    return lax.dot_general(a, b, (((0,), (0,)), ((), ())),
                           preferred_element_type=F32)


def _onehot(dstv, nrows):
    return (lax.broadcasted_iota(jnp.int32, (nrows, NF), 1) == dstv
            ).astype(F32)


_SMALL = pl.BlockSpec((NF, NF), lambda i: (0, 0))
_ROW = pl.BlockSpec((1, NF), lambda i: (0, 0))
_COL = pl.BlockSpec((NF, 1), lambda i: (0, 0))


def _payload(m):
    return jnp.concatenate(
        [m, jnp.ones((m.shape[0], 1), F32), jnp.zeros((m.shape[0], 63), F32)],
        axis=1)


def _tc_pass0(ev2, dst2, A0, wmf_e0, t2b, wmo_e0):
    def body(ev_r, dst_r, a0_r, wmfe_r, t2b_r, wmoe_r, mof_r, aggf_r, cnt_r):
        i = pl.program_id(0)
        ev = ev_r[...]
        oh = _onehot(dst_r[...], BE)
        mof_r[...] = _payload(
            jnp.maximum(_dot(oh, a0_r[...]) + ev * wmfe_r[...], 0.0))
        mfo = jnp.maximum(t2b_r[...] + ev * wmoe_r[...], 0.0)
        part = _dott(oh, mfo)
        cpart = _dott(oh, jnp.ones((BE, 1), F32))

        @pl.when(i == 0)
        def _():
            aggf_r[...] = jnp.zeros_like(aggf_r)
            cnt_r[...] = jnp.zeros_like(cnt_r)

        aggf_r[...] += part
        cnt_r[...] += cpart

    return pl.pallas_call(
        body,
        grid=(GE,),
        in_specs=[pl.BlockSpec((BE, 1), lambda i: (i, 0)),
                  pl.BlockSpec((BE, 1), lambda i: (i, 0)),
                  _SMALL, _ROW, _ROW, _ROW],
        out_specs=[pl.BlockSpec((BE, 2 * NF), lambda i: (i, 0)),
                   _SMALL, _COL],
        out_shape=[jax.ShapeDtypeStruct((E, 2 * NF), F32),
                   jax.ShapeDtypeStruct((NF, NF), F32),
                   jax.ShapeDtypeStruct((NF, 1), F32)],
    )(ev2, dst2, A0, wmf_e0, t2b, wmo_e0)


def _tc_node0(sums, wn_a0, vn0, we_n0, wmo_n1,
              aggf, cntd, f0b, wf_a0, we_f0, be0, wmf_f1, bmf1):
    def body(s_r, wna_r, vn0_r, wen_r, wmon_r,
             aggf_r, cntd_r, f0b_r, wfa_r, wef_r, be0_r, wmff_r, bmf1_r,
             nemb1_r, tab_r, inv_r, femb1_r, uf0b_r, tf1b_r):
        i = pl.program_id(0)
        sfull = s_r[0] + s_r[1]
        ssum = sfull[:, :NF]
        cnt = jnp.sum(sfull[:, NF:], axis=1, keepdims=True)
        inv = 1.0 / jnp.maximum(cnt, 1.0)
        inv_r[...] = inv
        agg = ssum * inv
        nn0 = jnp.maximum(vn0_r[...] + _dot(agg, wna_r[...]), 0.0)
        nemb1 = jnp.maximum(nn0 + 1.0, 0.0)
        nemb1_r[...] = nemb1
        tab_r[...] = jnp.concatenate(
            [_dot(nn0, wen_r[...]), _dot(nemb1, wmon_r[...])], axis=1)

        @pl.when(i == 0)
        def _():
            invd = 1.0 / jnp.maximum(cntd_r[...], 1.0)
            aggf0 = aggf_r[...] * invd
            nf0 = jnp.maximum(f0b_r[...] + _dot(aggf0, wfa_r[...]), 0.0)
            ey = (lax.broadcasted_iota(jnp.int32, (NF, NF), 0)
                  == lax.broadcasted_iota(jnp.int32, (NF, NF), 1)).astype(F32)
            femb1 = jnp.maximum(nf0 + ey, 0.0)
            femb1_r[...] = femb1
            uf0b_r[...] = _dot(nf0, wef_r[...]) + be0_r[...]
            tf1b_r[...] = _dot(femb1, wmff_r[...]) + bmf1_r[...]

    return pl.pallas_call(
        body,
        grid=(GN,),
        in_specs=[pl.BlockSpec((NC, BN, 2 * NF), lambda i: (0, i, 0)),
                  _SMALL, _ROW, _SMALL, _SMALL,
                  _SMALL, _COL, _SMALL, _SMALL, _SMALL, _ROW, _SMALL, _ROW],
        out_specs=[pl.BlockSpec((BN, NF), lambda i: (i, 0)),
                   pl.BlockSpec((BN, 2 * NF), lambda i: (i, 0)),
                   pl.BlockSpec((BN, 1), lambda i: (i, 0)),
                   _SMALL, _SMALL, _SMALL],
        out_shape=[jax.ShapeDtypeStruct((N, NF), F32),
                   jax.ShapeDtypeStruct((N, 2 * NF), F32),
                   jax.ShapeDtypeStruct((N, 1), F32),
                   jax.ShapeDtypeStruct((NF, NF), F32),
                   jax.ShapeDtypeStruct((NF, NF), F32),
                   jax.ShapeDtypeStruct((NF, NF), F32)],
    )(sums, wn_a0, vn0, we_n0, wmo_n1,
      aggf, cntd, f0b, wf_a0, we_f0, be0, wmf_f1, bmf1)


def _tc_pass1(ev2, dst2, g, uf0b, tf1b, wmf_e1, wmo_e1, bmo1, we_e0):
    def body(ev_r, dst_r, g_r, uf_r, tf_r, wmf_r, wmo_r, bmo_r,
             wee_r, e1_r, mof_r, aggf_r):
        i = pl.program_id(0)
        ev = ev_r[...]
        oh = _onehot(dst_r[...], BE)
        gv = g_r[...]
        inner = ev * wee_r[...] + gv[:, :NF] + _dot(oh, uf_r[...])
        e1 = jnp.maximum(jnp.maximum(inner, 0.0) + ev, 0.0)
        e1_r[...] = e1
        mof_r[...] = _payload(
            jnp.maximum(_dot(oh, tf_r[...]) + _dot(e1, wmf_r[...]), 0.0))
        mfo = jnp.maximum(gv[:, NF:] + _dot(e1, wmo_r[...]) + bmo_r[...], 0.0)
        part = _dott(oh, mfo)

        @pl.when(i == 0)
        def _():
            aggf_r[...] = jnp.zeros_like(aggf_r)

        aggf_r[...] += part

    return pl.pallas_call(
        body,
        grid=(GE,),
        in_specs=[pl.BlockSpec((BE, 1), lambda i: (i, 0)),
                  pl.BlockSpec((BE, 1), lambda i: (i, 0)),
                  pl.BlockSpec((BE, 2 * NF), lambda i: (i, 0)),
                  _SMALL, _SMALL, _SMALL, _SMALL, _ROW, _ROW],
        out_specs=[pl.BlockSpec((BE, NF), lambda i: (i, 0)),
                   pl.BlockSpec((BE, 2 * NF), lambda i: (i, 0)),
                   _SMALL],
        out_shape=[jax.ShapeDtypeStruct((E, NF), F32),
                   jax.ShapeDtypeStruct((E, 2 * NF), F32),
                   jax.ShapeDtypeStruct((NF, NF), F32)],
    )(ev2, dst2, g, uf0b, tf1b, wmf_e1, wmo_e1, bmo1, we_e0)


def _tc_node1(nemb1, sums, inv, wn_n1, wn_a1, bn1, we_n1, wmo_n2,
              femb1, aggf, cntd, wf_f1, wf_a1, bf1, we_f1, be1, wmf_f2, bmf2):
    def body(ne_r, s_r, inv_r, wnn_r, wna_r, bn_r, wen_r, wmon_r,
             fe_r, aggf_r, cntd_r, wff_r, wfa_r, bf_r, wef_r, be_r,
             wmff_r, bmf_r,
             nemb2_r, tab_r, femb2_r, uf1b_r, tf2b_r):
        i = pl.program_id(0)
        ne = ne_r[...]
        agg = (s_r[0] + s_r[1])[:, :NF] * inv_r[...]
        nn1 = jnp.maximum(_dot(ne, wnn_r[...]) + _dot(agg, wna_r[...])
                          + bn_r[...], 0.0)
        nemb2 = jnp.maximum(nn1 + ne, 0.0)
        nemb2_r[...] = nemb2
        tab_r[...] = jnp.concatenate(
            [_dot(nn1, wen_r[...]), _dot(nemb2, wmon_r[...])], axis=1)

        @pl.when(i == 0)
        def _():
            fe = fe_r[...]
            invd = 1.0 / jnp.maximum(cntd_r[...], 1.0)
            aggf1 = aggf_r[...] * invd
            nf1 = jnp.maximum(_dot(fe, wff_r[...]) + _dot(aggf1, wfa_r[...])
                              + bf_r[...], 0.0)
            femb2 = jnp.maximum(nf1 + fe, 0.0)
            femb2_r[...] = femb2
            uf1b_r[...] = _dot(nf1, wef_r[...]) + be_r[...]
            tf2b_r[...] = _dot(femb2, wmff_r[...]) + bmf_r[...]

    return pl.pallas_call(
        body,
        grid=(GN,),
        in_specs=[pl.BlockSpec((BN, NF), lambda i: (i, 0)),
                  pl.BlockSpec((NC, BN, 2 * NF), lambda i: (0, i, 0)),
                  pl.BlockSpec((BN, 1), lambda i: (i, 0)),
                  _SMALL, _SMALL, _ROW, _SMALL, _SMALL,
                  _SMALL, _SMALL, _COL, _SMALL, _SMALL, _ROW, _SMALL, _ROW,
                  _SMALL, _ROW],
        out_specs=[pl.BlockSpec((BN, NF), lambda i: (i, 0)),
                   pl.BlockSpec((BN, 2 * NF), lambda i: (i, 0)),
                   _SMALL, _SMALL, _SMALL],
        out_shape=[jax.ShapeDtypeStruct((N, NF), F32),
                   jax.ShapeDtypeStruct((N, 2 * NF), F32),
                   jax.ShapeDtypeStruct((NF, NF), F32),
                   jax.ShapeDtypeStruct((NF, NF), F32),
                   jax.ShapeDtypeStruct((NF, NF), F32)],
    )(nemb1, sums, inv, wn_n1, wn_a1, bn1, we_n1, wmo_n2,
      femb1, aggf, cntd, wf_f1, wf_a1, bf1, we_f1, be1, wmf_f2, bmf2)


def _tc_pass2(e1, dst2, g, uf1b, tf2b, we_e1, wmf_e2, wmo_e2, bmo2):
    def body(e1_r, dst_r, g_r, uf_r, tf_r, wee_r, wmf_r, wmo_r,
             bmo_r, mof_r, aggf_r):
        i = pl.program_id(0)
        e1v = e1_r[...]
        oh = _onehot(dst_r[...], BE)
        gv = g_r[...]
        inner = _dot(e1v, wee_r[...]) + gv[:, :NF] + _dot(oh, uf_r[...])
        e2 = jnp.maximum(jnp.maximum(inner, 0.0) + e1v, 0.0)
        mof_r[...] = _payload(
            jnp.maximum(_dot(oh, tf_r[...]) + _dot(e2, wmf_r[...]), 0.0))
        mfo = jnp.maximum(gv[:, NF:] + _dot(e2, wmo_r[...]) + bmo_r[...], 0.0)
        part = _dott(oh, mfo)

        @pl.when(i == 0)
        def _():
            aggf_r[...] = jnp.zeros_like(aggf_r)

        aggf_r[...] += part

    return pl.pallas_call(
        body,
        grid=(GE,),
        in_specs=[pl.BlockSpec((BE, NF), lambda i: (i, 0)),
                  pl.BlockSpec((BE, 1), lambda i: (i, 0)),
                  pl.BlockSpec((BE, 2 * NF), lambda i: (i, 0)),
                  _SMALL, _SMALL, _SMALL, _SMALL, _SMALL, _ROW],
        out_specs=[pl.BlockSpec((BE, 2 * NF), lambda i: (i, 0)),
                   _SMALL],
        out_shape=[jax.ShapeDtypeStruct((E, 2 * NF), F32),
                   jax.ShapeDtypeStruct((NF, NF), F32)],
    )(e1, dst2, g, uf1b, tf2b, we_e1, wmf_e2, wmo_e2, bmo2)


def _tc_node2_head(nemb2, sums, inv, wn_n2, wn_a2, bn2,
                   femb2, aggf, cntd, wf_f2, wf_a2, bf2,
                   wo, wfe, bh, wout, bout, w1, b1, w2, b2):
    def body(ne_r, s_r, inv_r, wnn_r, wna_r, bn_r,
             fe_r, aggf_r, cntd_r, wff_r, wfa_r, bf_r,
             wo_r, wfe_r, bh_r, wout_r, bout_r, w1_r, b1_r, w2_r, b2_r,
             dhat_r, yhat_r):
        ne = ne_r[...]
        agg = (s_r[0] + s_r[1])[:, :NF] * inv_r[...]
        nn2 = jnp.maximum(_dot(ne, wnn_r[...]) + _dot(agg, wna_r[...])
                          + bn_r[...], 0.0)
        nemb3 = jnp.maximum(nn2 + ne, 0.0)
        obs_h = _dot(nemb3, wo_r[...])
        fe = fe_r[...]
        invd = 1.0 / jnp.maximum(cntd_r[...], 1.0)
        aggf2 = aggf_r[...] * invd
        nf2 = jnp.maximum(_dot(fe, wff_r[...]) + _dot(aggf2, wfa_r[...])
                          + bf_r[...], 0.0)
        femb3 = jnp.maximum(nf2 + fe, 0.0)
        cmat = _dot(femb3, wfe_r[...]) + bh_r[...]
        h3 = jnp.maximum(obs_h[:, None, :] + cmat[None, :, :], 0.0)
        d = (_dot(h3.reshape(BH * NF, NF), wout_r[...]).reshape(BH, NF)
             + bout_r[...])
        dhat_r[...] = d
        y = (_dot(jnp.maximum(_dot(d, w1_r[...]) + b1_r[...], 0.0), w2_r[...])
             + b2_r[...])
        yhat_r[...] = y

    return pl.pallas_call(
        body,
        grid=(GH,),
        in_specs=[pl.BlockSpec((BH, NF), lambda i: (i, 0)),
                  pl.BlockSpec((NC, BH, 2 * NF), lambda i: (0, i, 0)),
                  pl.BlockSpec((BH, 1), lambda i: (i, 0)),
                  _SMALL, _SMALL, _ROW,
                  _SMALL, _SMALL, _COL, _SMALL, _SMALL, _ROW,
                  _SMALL, _SMALL, _ROW, _COL, pl.BlockSpec((1, 1), lambda i: (0, 0)),
                  _SMALL, _ROW, _COL, pl.BlockSpec((1, 1), lambda i: (0, 0))],
        out_specs=[pl.BlockSpec((BH, NF), lambda i: (i, 0)),
                   pl.BlockSpec((BH, 1), lambda i: (i, 0))],
        out_shape=[jax.ShapeDtypeStruct((N, NF), F32),
                   jax.ShapeDtypeStruct((N, 1), F32)],
    )(nemb2, sums, inv, wn_n2, wn_a2, bn2,
      femb2, aggf, cntd, wf_f2, wf_a2, bf2,
      wo, wfe, bh, wout, bout, w1, b1, w2, b2)


def kernel(x, edge_index, edge_value, params):
    del x
    src = edge_index[0]
    dst = edge_index[1]
    ev2 = edge_value.reshape(E, 1)
    dst2 = dst.reshape(E, 1)
    src2d = src
    zeros128 = jnp.zeros((N, 128), F32)

    p0, p1, p2 = params["block0"], params["block1"], params["block2"]
    eph, nph = params["eph"], params["nph"]

    def row(v):
        return v.reshape(1, NF)

    A0 = p0["Wmf"][:64] + row(p0["bmf"])
    wmf_e0 = p0["Wmf"][64:65]
    t2b = jnp.sum(p0["Wmo"][:64], 0, keepdims=True) + row(p0["bmo"])
    wmo_e0 = p0["Wmo"][64:65]
    vn0 = jnp.sum(p0["Wn"][:64], 0, keepdims=True) + row(p0["bn"])
    wn_a0 = p0["Wn"][64:]
    f0b = p0["Wf"][:64] + row(p0["bf"])
    wf_a0 = p0["Wf"][64:]
    we_e0 = p0["We"][0:1]
    we_n0 = p0["We"][1:65]
    we_f0 = p0["We"][65:]
    be0 = row(p0["be"])

    wmf_f1, wmf_e1 = p1["Wmf"][:64], p1["Wmf"][64:]
    wmo_n1, wmo_e1 = p1["Wmo"][:64], p1["Wmo"][64:]
    wn_n1, wn_a1 = p1["Wn"][:64], p1["Wn"][64:]
    wf_f1, wf_a1 = p1["Wf"][:64], p1["Wf"][64:]
    we_e1, we_n1, we_f1 = p1["We"][:64], p1["We"][64:128], p1["We"][128:]
    bmf1, bmo1, bn1, bf1, be1 = (row(p1["bmf"]), row(p1["bmo"]),
                                 row(p1["bn"]), row(p1["bf"]), row(p1["be"]))

    wmf_f2, wmf_e2 = p2["Wmf"][:64], p2["Wmf"][64:]
    wmo_n2, wmo_e2 = p2["Wmo"][:64], p2["Wmo"][64:]
    wn_n2, wn_a2 = p2["Wn"][:64], p2["Wn"][64:]
    wf_f2, wf_a2 = p2["Wf"][:64], p2["Wf"][64:]
    bmf2, bmo2, bn2, bf2 = (row(p2["bmf"]), row(p2["bmo"]),
                            row(p2["bn"]), row(p2["bf"]))

    mof0, aggf0, cntd = _tc_pass0(ev2, dst2, A0, wmf_e0, t2b, wmo_e0)
    sums0 = _sc_scatter(mof0, src2d, zeros128)

    (nemb1, tab1, inv, femb1, uf0b, tf1b) = _tc_node0(
        sums0, wn_a0, vn0, we_n0, wmo_n1,
        aggf0, cntd, f0b, wf_a0, we_f0, be0, wmf_f1, bmf1)

    g0 = _sc_gather(tab1, src2d)
    e1, mof1, aggf1 = _tc_pass1(ev2, dst2, g0, uf0b, tf1b,
                                wmf_e1, wmo_e1, bmo1, we_e0)
    sums1 = _sc_scatter(mof1, src2d, zeros128)

    (nemb2, tab2, femb2, uf1b, tf2b) = _tc_node1(
        nemb1, sums1, inv, wn_n1, wn_a1, bn1, we_n1, wmo_n2,
        femb1, aggf1, cntd, wf_f1, wf_a1, bf1, we_f1, be1, wmf_f2, bmf2)

    g1 = _sc_gather(tab2, src2d)
    mof2, aggf2 = _tc_pass2(e1, dst2, g1, uf1b, tf2b,
                            we_e1, wmf_e2, wmo_e2, bmo2)
    sums2 = _sc_scatter(mof2, src2d, zeros128)

    d_hat, y_hat = _tc_node2_head(
        nemb2, sums2, inv, wn_n2, wn_a2, bn2,
        femb2, aggf2, cntd, wf_f2, wf_a2, bf2,
        eph["Wo"], eph["Wf"], row(eph["bh"]), eph["wout"].reshape(NF, 1),
        eph["bout"].reshape(1, 1), nph["W1"], row(nph["b1"]),
        nph["W2"], nph["b2"].reshape(1, 1))
    return d_hat, y_hat
